# baseline (speedup 1.0000x reference)
import functools

import jax
import jax.numpy as jnp
from jax import lax
from jax.experimental import pallas as pl
from jax.experimental.pallas import tpu as pltpu
from jax.experimental.pallas import tpu_sc as plsc

N = 10000
E = 160000
F = 128
NC = 2
NS = 16
EB = 128
NB = E // EB
NBP = 1280
EP = NBP * EB
NROWS = 10016
NDEG = 10240
BN = 2000


def _make_deg():
    mesh = plsc.VectorSubcoreMesh(core_axis_name="c", subcore_axis_name="s")
    per_tile = NDEG // NS

    @functools.partial(
        pl.kernel,
        out_type=jax.ShapeDtypeStruct((NC * NDEG,), jnp.float32),
        mesh=mesh,
        scratch_types=[
            pltpu.VMEM((EB,), jnp.int32),
            pltpu.VMEM((EB,), jnp.float32),
            pltpu.VMEM((per_tile,), jnp.float32),
            pltpu.VMEM_SHARED((NDEG,), jnp.float32),
        ],
    )
    def deg_kernel(dst_hbm, out_hbm, dst_v, ones_v, zeros_v, acc):
        c = lax.axis_index("c")
        s = lax.axis_index("s")
        w = s * NC + c
        for i in range(EB // 16):
            ones_v[pl.ds(i * 16, 16)] = jnp.ones((16,), jnp.float32)
        for i in range(per_tile // 16):
            zeros_v[pl.ds(i * 16, 16)] = jnp.zeros((16,), jnp.float32)
        pltpu.sync_copy(zeros_v, acc.at[pl.ds(s * per_tile, per_tile)])
        plsc.subcore_barrier()
        nb = jnp.where(w < NB % 32, NB // 32 + 1, NB // 32)

        def body(i, carry):
            b = w + i * 32
            pltpu.sync_copy(dst_hbm.at[pl.ds(b * EB, EB)], dst_v)
            pltpu.sync_copy(ones_v, acc.at[dst_v], add=True)
            return carry

        lax.fori_loop(0, nb, body, 0)
        plsc.subcore_barrier()
        pltpu.sync_copy(
            acc.at[pl.ds(s * per_tile, per_tile)],
            out_hbm.at[pl.ds(c * NDEG + s * per_tile, per_tile)],
        )

    return deg_kernel


def _make_prop(C):
    cpc = C // NC
    mesh = plsc.VectorSubcoreMesh(core_axis_name="c", subcore_axis_name="s")
    rpt = 624
    rem = N - NS * rpt
    bpt = NBP // NS

    @functools.partial(
        pl.kernel,
        out_type=jax.ShapeDtypeStruct((C * N, F), jnp.float32),
        mesh=mesh,
        scratch_types=[
            pltpu.VMEM((bpt, EB), jnp.int32),
            pltpu.VMEM((bpt, EB), jnp.int32),
            pltpu.VMEM((bpt, EB), jnp.int32),
            pltpu.VMEM((EB, F), jnp.float32),
            pltpu.VMEM_SHARED((NROWS, F), jnp.float32),
        ],
    )
    def prop_kernel(hs_hbm, src_hbm, dst_hbm, out_hbm,
                    src_all, adj_all, dst_all, rows, acc):
        c = lax.axis_index("c")
        s = lax.axis_index("s")
        q0 = s * bpt
        pltpu.sync_copy(src_hbm.at[pl.ds(q0, bpt)], src_all)
        pltpu.sync_copy(dst_hbm.at[pl.ds(q0, bpt)], dst_all)

        for r in range(cpc):
            chunk = c * cpc + r
            row0 = chunk * N
            pltpu.sync_copy(
                hs_hbm.at[pl.ds(row0 + s * rpt, rpt)],
                acc.at[pl.ds(s * rpt, rpt)],
            )

            @pl.when(s == NS - 1)
            def _():
                pltpu.sync_copy(
                    hs_hbm.at[pl.ds(row0 + NS * rpt, rem)],
                    acc.at[pl.ds(NS * rpt, rem)],
                )

            off = lax.broadcast(row0, (16,))

            def adj_body(b, carry):
                for i2 in range(EB // 16):
                    adj_all[b, pl.ds(i2 * 16, 16)] = (
                        src_all[b, pl.ds(i2 * 16, 16)] + off)
                return carry

            lax.fori_loop(0, bpt, adj_body, 0)
            plsc.subcore_barrier()

            def body(t, carry):
                pltpu.sync_copy(hs_hbm.at[adj_all.at[t]], rows)
                pltpu.sync_copy(rows, acc.at[dst_all.at[t]], add=True)
                return carry

            lax.fori_loop(0, bpt, body, 0)

            plsc.subcore_barrier()
            pltpu.sync_copy(
                acc.at[pl.ds(s * rpt, rpt)],
                out_hbm.at[pl.ds(row0 + s * rpt, rpt)],
            )

            @pl.when(s == NS - 1)
            def _():
                pltpu.sync_copy(
                    acc.at[pl.ds(NS * rpt, rem)],
                    out_hbm.at[pl.ds(row0 + NS * rpt, rem)],
                )

            if r != cpc - 1:
                plsc.subcore_barrier()

    return prop_kernel


_deg_call = _make_deg()
_prop4_call = _make_prop(4)
_prop2_call = _make_prop(2)


def _dis_body(degp_ref, dis_ref):
    d = 1.0 + degp_ref[0:NDEG // F, :] + degp_ref[NDEG // F:, :]
    dis_ref[...] = lax.rsqrt(d)


def _dis_call(degp):
    out = pl.pallas_call(
        _dis_body,
        out_shape=jax.ShapeDtypeStruct((NDEG // F, F), jnp.float32),
    )(degp.reshape(2 * NDEG // F, F))
    return out.reshape(NDEG)[:N].reshape(N, 1)


def _mm_scale_body(x_ref, w_ref, dis_ref, out_ref):
    h = jnp.dot(x_ref[...], w_ref[...], preferred_element_type=jnp.float32)
    out_ref[...] = h * dis_ref[...]


def _mm_scale_call(x, W, dis2d, C):
    k = x.shape[1]
    return pl.pallas_call(
        _mm_scale_body,
        grid=(N // BN, C),
        in_specs=[
            pl.BlockSpec((BN, k), lambda n, c: (n, 0)),
            pl.BlockSpec((k, F), lambda n, c: (0, c)),
            pl.BlockSpec((BN, 1), lambda n, c: (n, 0)),
        ],
        out_specs=pl.BlockSpec((BN, F), lambda n, c: (c * (N // BN) + n, 0)),
        out_shape=jax.ShapeDtypeStruct((C * N, F), jnp.float32),
    )(x, W, dis2d)


def _mid_body(p_ref, b_ref, w_ref, dis_ref, out_ref, *, nk):
    k = pl.program_id(2)

    @pl.when(k == 0)
    def _():
        out_ref[...] = jnp.zeros_like(out_ref)

    t = jnp.maximum(p_ref[...] * dis_ref[...] + b_ref[0], 0.0)
    out_ref[...] += jnp.dot(t, w_ref[...], preferred_element_type=jnp.float32)

    @pl.when(k == nk - 1)
    def _():
        out_ref[...] *= dis_ref[...]


def _mid_call(p1, b1r, W2, dis2d, C_in, C_out):
    return pl.pallas_call(
        functools.partial(_mid_body, nk=C_in),
        grid=(N // BN, C_out, C_in),
        in_specs=[
            pl.BlockSpec((BN, F), lambda n, f, k: (k * (N // BN) + n, 0)),
            pl.BlockSpec((1, 1, F), lambda n, f, k: (k, 0, 0)),
            pl.BlockSpec((F, F), lambda n, f, k: (k, f)),
            pl.BlockSpec((BN, 1), lambda n, f, k: (n, 0)),
        ],
        out_specs=pl.BlockSpec((BN, F), lambda n, f, k: (f * (N // BN) + n, 0)),
        out_shape=jax.ShapeDtypeStruct((C_out * N, F), jnp.float32),
    )(p1, b1r, W2, dis2d)


def _final_body(p_ref, b_ref, dis_ref, out_ref):
    out_ref[...] = p_ref[...] * dis_ref[...] + b_ref[0]


def _final_call(p2, b2r, dis2d, C):
    return pl.pallas_call(
        _final_body,
        grid=(N // BN, C),
        in_specs=[
            pl.BlockSpec((BN, F), lambda n, f: (f * (N // BN) + n, 0)),
            pl.BlockSpec((1, 1, F), lambda n, f: (f, 0, 0)),
            pl.BlockSpec((BN, 1), lambda n, f: (n, 0)),
        ],
        out_specs=pl.BlockSpec((BN, F), lambda n, f: (n, f)),
        out_shape=jax.ShapeDtypeStruct((N, C * F), jnp.float32),
    )(p2, b2r, dis2d)


def kernel(x, edge_index, W1, b1, W2, b2):
    src = edge_index[0].astype(jnp.int32)
    dst = edge_index[1].astype(jnp.int32)
    src_p = jnp.concatenate(
        [src, jnp.zeros(EP - E, jnp.int32)]).reshape(NBP, EB)
    dst_p = jnp.concatenate(
        [dst, jnp.full(EP - E, N, jnp.int32)]).reshape(NBP, EB)

    degp = _deg_call(dst)
    dis2d = _dis_call(degp)

    hs1 = _mm_scale_call(x, W1, dis2d, 4)
    p1 = _prop4_call(hs1, src_p, dst_p)
    hs2 = _mid_call(p1, b1.reshape(4, 1, F), W2, dis2d, 4, 2)
    p2 = _prop2_call(hs2, src_p, dst_p)
    z = _final_call(p2, b2.reshape(2, 1, F), dis2d, 2)
    return z


# speedup vs baseline: 1.1116x; 1.1116x over previous
import functools

import jax
import jax.numpy as jnp
from jax import lax
from jax.experimental import pallas as pl
from jax.experimental.pallas import tpu as pltpu
from jax.experimental.pallas import tpu_sc as plsc

N = 10000
E = 160000
F = 128
NC = 2
NS = 16
EB = 128
NB = E // EB
NBP = 1280
EP = NBP * EB
NROWS = 10016
NDEG = 10240
BN = 2000


def _make_deg():
    mesh = plsc.VectorSubcoreMesh(core_axis_name="c", subcore_axis_name="s")
    per_tile = NDEG // NS

    @functools.partial(
        pl.kernel,
        out_type=jax.ShapeDtypeStruct((NC * NDEG,), jnp.float32),
        mesh=mesh,
        scratch_types=[
            pltpu.VMEM((EB,), jnp.int32),
            pltpu.VMEM((EB,), jnp.float32),
            pltpu.VMEM((per_tile,), jnp.float32),
            pltpu.VMEM_SHARED((NDEG,), jnp.float32),
        ],
    )
    def deg_kernel(dst_hbm, out_hbm, dst_v, ones_v, zeros_v, acc):
        c = lax.axis_index("c")
        s = lax.axis_index("s")
        w = s * NC + c
        for i in range(EB // 16):
            ones_v[pl.ds(i * 16, 16)] = jnp.ones((16,), jnp.float32)
        for i in range(per_tile // 16):
            zeros_v[pl.ds(i * 16, 16)] = jnp.zeros((16,), jnp.float32)
        pltpu.sync_copy(zeros_v, acc.at[pl.ds(s * per_tile, per_tile)])
        plsc.subcore_barrier()
        nb = jnp.where(w < NB % 32, NB // 32 + 1, NB // 32)

        def body(i, carry):
            b = w + i * 32
            pltpu.sync_copy(dst_hbm.at[pl.ds(b * EB, EB)], dst_v)
            pltpu.sync_copy(ones_v, acc.at[dst_v], add=True)
            return carry

        lax.fori_loop(0, nb, body, 0)
        plsc.subcore_barrier()
        pltpu.sync_copy(
            acc.at[pl.ds(s * per_tile, per_tile)],
            out_hbm.at[pl.ds(c * NDEG + s * per_tile, per_tile)],
        )

    return deg_kernel


def _make_prop(C):
    cpc = C // NC
    mesh = plsc.VectorSubcoreMesh(core_axis_name="c", subcore_axis_name="s")
    rpt = 624
    rem = N - NS * rpt
    bpt = NBP // NS
    HB = bpt // 2

    @functools.partial(
        pl.kernel,
        out_type=jax.ShapeDtypeStruct((C * N, F), jnp.float32),
        mesh=mesh,
        scratch_types=[
            pltpu.VMEM((HB, EB), jnp.int32),
            pltpu.VMEM((bpt, EB), jnp.int32),
            [pltpu.VMEM((EB, F), jnp.float32) for _ in range(2)],
            pltpu.VMEM_SHARED((NROWS, F), jnp.float32),
            [pltpu.SemaphoreType.DMA for _ in range(2)],
        ],
    )
    def prop_kernel(hs_hbm, src_hbm, dst_hbm, out_hbm,
                    src_h, dst_all, rows, acc, ssem):
        c = lax.axis_index("c")
        s = lax.axis_index("s")
        q0 = s * bpt
        pltpu.sync_copy(dst_hbm.at[pl.ds(q0, bpt)], dst_all)

        def s_wait(u):
            pltpu.make_async_copy(
                rows[u], acc.at[dst_all.at[0]], ssem[u]).wait()

        for r in range(cpc):
            chunk = c * cpc + r
            row0 = chunk * N
            pltpu.sync_copy(
                hs_hbm.at[pl.ds(row0 + s * rpt, rpt)],
                acc.at[pl.ds(s * rpt, rpt)],
            )

            @pl.when(s == NS - 1)
            def _():
                pltpu.sync_copy(
                    hs_hbm.at[pl.ds(row0 + NS * rpt, rem)],
                    acc.at[pl.ds(NS * rpt, rem)],
                )

            plsc.subcore_barrier()

            def step(t, u, h, first=False):
                if not first:
                    s_wait(u)
                pltpu.sync_copy(hs_hbm.at[src_h.at[t]], rows[u])
                pltpu.async_copy(rows[u], acc.at[dst_all.at[h * HB + t]],
                                 ssem[u], add=True)

            for h in range(2):
                pltpu.sync_copy(
                    src_hbm.at[pl.ds(chunk * NBP + q0 + h * HB, HB)], src_h)

                if h == 0:
                    step(0, 0, 0, first=True)
                    step(1, 1, 0, first=True)

                def pair(j, carry):
                    for k in range(2):
                        if h == 0:
                            step(2 + 2 * j + k, k, 0)
                        else:
                            step(2 * j + k, k, 1)
                    return carry

                lax.fori_loop(0, (HB - 2) // 2 if h == 0 else HB // 2,
                              pair, 0)

            s_wait(0)
            s_wait(1)
            plsc.subcore_barrier()
            pltpu.sync_copy(
                acc.at[pl.ds(s * rpt, rpt)],
                out_hbm.at[pl.ds(row0 + s * rpt, rpt)],
            )

            @pl.when(s == NS - 1)
            def _():
                pltpu.sync_copy(
                    acc.at[pl.ds(NS * rpt, rem)],
                    out_hbm.at[pl.ds(row0 + NS * rpt, rem)],
                )

            if r != cpc - 1:
                plsc.subcore_barrier()

    return prop_kernel


_deg_call = _make_deg()
_prop4_call = _make_prop(4)
_prop2_call = _make_prop(2)


def _dis_body(degp_ref, dis_ref):
    d = 1.0 + degp_ref[0:NDEG // F, :] + degp_ref[NDEG // F:, :]
    dis_ref[...] = lax.rsqrt(d)


def _dis_call(degp):
    out = pl.pallas_call(
        _dis_body,
        out_shape=jax.ShapeDtypeStruct((NDEG // F, F), jnp.float32),
    )(degp.reshape(2 * NDEG // F, F))
    return out.reshape(NDEG)[:N].reshape(N, 1)


def _mm_scale_body(x_ref, w_ref, dis_ref, out_ref):
    h = jnp.dot(x_ref[...], w_ref[...], preferred_element_type=jnp.float32)
    out_ref[...] = h * dis_ref[...]


def _mm_scale_call(x, W, dis2d, C):
    k = x.shape[1]
    return pl.pallas_call(
        _mm_scale_body,
        grid=(N // BN, C),
        in_specs=[
            pl.BlockSpec((BN, k), lambda n, c: (n, 0)),
            pl.BlockSpec((k, F), lambda n, c: (0, c)),
            pl.BlockSpec((BN, 1), lambda n, c: (n, 0)),
        ],
        out_specs=pl.BlockSpec((BN, F), lambda n, c: (c * (N // BN) + n, 0)),
        out_shape=jax.ShapeDtypeStruct((C * N, F), jnp.float32),
    )(x, W, dis2d)


def _mid_body(p_ref, b_ref, w_ref, dis_ref, out_ref, *, nk):
    k = pl.program_id(2)

    @pl.when(k == 0)
    def _():
        out_ref[...] = jnp.zeros_like(out_ref)

    t = jnp.maximum(p_ref[...] * dis_ref[...] + b_ref[0], 0.0)
    out_ref[...] += jnp.dot(t, w_ref[...], preferred_element_type=jnp.float32)

    @pl.when(k == nk - 1)
    def _():
        out_ref[...] *= dis_ref[...]


def _mid_call(p1, b1r, W2, dis2d, C_in, C_out):
    return pl.pallas_call(
        functools.partial(_mid_body, nk=C_in),
        grid=(N // BN, C_out, C_in),
        in_specs=[
            pl.BlockSpec((BN, F), lambda n, f, k: (k * (N // BN) + n, 0)),
            pl.BlockSpec((1, 1, F), lambda n, f, k: (k, 0, 0)),
            pl.BlockSpec((F, F), lambda n, f, k: (k, f)),
            pl.BlockSpec((BN, 1), lambda n, f, k: (n, 0)),
        ],
        out_specs=pl.BlockSpec((BN, F), lambda n, f, k: (f * (N // BN) + n, 0)),
        out_shape=jax.ShapeDtypeStruct((C_out * N, F), jnp.float32),
    )(p1, b1r, W2, dis2d)


def _final_body(p_ref, b_ref, dis_ref, out_ref):
    out_ref[...] = p_ref[...] * dis_ref[...] + b_ref[0]


def _final_call(p2, b2r, dis2d, C):
    return pl.pallas_call(
        _final_body,
        grid=(N // BN, C),
        in_specs=[
            pl.BlockSpec((BN, F), lambda n, f: (f * (N // BN) + n, 0)),
            pl.BlockSpec((1, 1, F), lambda n, f: (f, 0, 0)),
            pl.BlockSpec((BN, 1), lambda n, f: (n, 0)),
        ],
        out_specs=pl.BlockSpec((BN, F), lambda n, f: (n, f)),
        out_shape=jax.ShapeDtypeStruct((N, C * F), jnp.float32),
    )(p2, b2r, dis2d)


def kernel(x, edge_index, W1, b1, W2, b2):
    src = edge_index[0].astype(jnp.int32)
    dst = edge_index[1].astype(jnp.int32)
    src_p = jnp.concatenate(
        [src, jnp.zeros(EP - E, jnp.int32)]).reshape(NBP, EB)
    dst_p = jnp.concatenate(
        [dst, jnp.full(EP - E, N, jnp.int32)]).reshape(NBP, EB)
    src_pc4 = (src_p[None, :, :]
               + (jnp.arange(4, dtype=jnp.int32) * N)[:, None, None]
               ).reshape(4 * NBP, EB)
    src_pc2 = src_pc4[:2 * NBP]

    degp = _deg_call(dst)
    dis2d = _dis_call(degp)

    hs1 = _mm_scale_call(x, W1, dis2d, 4)
    p1 = _prop4_call(hs1, src_pc4, dst_p)
    hs2 = _mid_call(p1, b1.reshape(4, 1, F), W2, dis2d, 4, 2)
    p2 = _prop2_call(hs2, src_pc2, dst_p)
    z = _final_call(p2, b2.reshape(2, 1, F), dis2d, 2)
    return z


# speedup vs baseline: 1.2436x; 1.1188x over previous
import functools

import jax
import jax.numpy as jnp
from jax import lax
from jax.experimental import pallas as pl
from jax.experimental.pallas import tpu as pltpu
from jax.experimental.pallas import tpu_sc as plsc

N = 10000
E = 160000
F = 128
NC = 2
NS = 16
EB = 128
NB = E // EB
NBP = 1280
EP = NBP * EB
NROWS = 10016
NDEG = 10240
BN = 2000


def _make_deg():
    mesh = plsc.VectorSubcoreMesh(core_axis_name="c", subcore_axis_name="s")
    per_tile = NDEG // NS

    @functools.partial(
        pl.kernel,
        out_type=jax.ShapeDtypeStruct((NC * NDEG,), jnp.float32),
        mesh=mesh,
        scratch_types=[
            pltpu.VMEM((EB,), jnp.int32),
            pltpu.VMEM((EB,), jnp.float32),
            pltpu.VMEM((per_tile,), jnp.float32),
            pltpu.VMEM_SHARED((NDEG,), jnp.float32),
        ],
    )
    def deg_kernel(dst_hbm, out_hbm, dst_v, ones_v, zeros_v, acc):
        c = lax.axis_index("c")
        s = lax.axis_index("s")
        w = s * NC + c
        for i in range(EB // 16):
            ones_v[pl.ds(i * 16, 16)] = jnp.ones((16,), jnp.float32)
        for i in range(per_tile // 16):
            zeros_v[pl.ds(i * 16, 16)] = jnp.zeros((16,), jnp.float32)
        pltpu.sync_copy(zeros_v, acc.at[pl.ds(s * per_tile, per_tile)])
        plsc.subcore_barrier()
        nb = jnp.where(w < NB % 32, NB // 32 + 1, NB // 32)

        def body(i, carry):
            b = w + i * 32
            pltpu.sync_copy(dst_hbm.at[pl.ds(b * EB, EB)], dst_v)
            pltpu.sync_copy(ones_v, acc.at[dst_v], add=True)
            return carry

        lax.fori_loop(0, nb, body, 0)
        plsc.subcore_barrier()
        pltpu.sync_copy(
            acc.at[pl.ds(s * per_tile, per_tile)],
            out_hbm.at[pl.ds(c * NDEG + s * per_tile, per_tile)],
        )

    return deg_kernel


def _make_prop(C):
    cpc = C // NC
    mesh = plsc.VectorSubcoreMesh(core_axis_name="c", subcore_axis_name="s")
    rpt = 624
    rem = N - NS * rpt
    bpt = NBP // NS
    HB = bpt // 2

    @functools.partial(
        pl.kernel,
        out_type=jax.ShapeDtypeStruct((C * N, F), jnp.float32),
        mesh=mesh,
        scratch_types=[
            [pltpu.VMEM((EB,), jnp.int32) for _ in range(3)],
            [pltpu.VMEM((EB,), jnp.int32) for _ in range(3)],
            [pltpu.VMEM((EB, F), jnp.float32) for _ in range(3)],
            pltpu.VMEM_SHARED((NROWS, F), jnp.float32),
            [pltpu.SemaphoreType.DMA for _ in range(3)],
            [pltpu.SemaphoreType.DMA for _ in range(3)],
            [pltpu.SemaphoreType.DMA for _ in range(3)],
            [pltpu.SemaphoreType.DMA for _ in range(3)],
        ],
    )
    def prop_kernel(hs_hbm, srcf, dstf, out_hbm,
                    sbuf, dbuf, rows, acc, gsem, ssem, sisem, disem):
        c = lax.axis_index("c")
        s = lax.axis_index("s")
        q0 = s * bpt
        db = q0 * EB

        for r in range(cpc):
            chunk = c * cpc + r
            row0 = chunk * N
            cb = (chunk * NBP + q0) * EB

            def ips(v, u):
                pltpu.async_copy(srcf.at[pl.ds(cb + v * EB, EB)],
                                 sbuf[u], sisem[u])

            def ipd(v, u):
                pltpu.async_copy(dstf.at[pl.ds(db + v * EB, EB)],
                                 dbuf[u], disem[u])

            def iws(u):
                pltpu.make_async_copy(srcf.at[pl.ds(0, EB)],
                                      sbuf[u], sisem[u]).wait()

            def iwd(u):
                pltpu.make_async_copy(dstf.at[pl.ds(0, EB)],
                                      dbuf[u], disem[u]).wait()

            def gis(u):
                pltpu.async_copy(hs_hbm.at[sbuf[u]], rows[u], gsem[u])

            def gw(u):
                pltpu.make_async_copy(hs_hbm.at[sbuf[u]],
                                      rows[u], gsem[u]).wait()

            def sis(u):
                pltpu.async_copy(rows[u], acc.at[dbuf[u]], ssem[u], add=True)

            def sw(u):
                pltpu.make_async_copy(rows[u], acc.at[dbuf[u]],
                                      ssem[u]).wait()

            pltpu.sync_copy(
                hs_hbm.at[pl.ds(row0 + s * rpt, rpt)],
                acc.at[pl.ds(s * rpt, rpt)],
            )

            @pl.when(s == NS - 1)
            def _():
                pltpu.sync_copy(
                    hs_hbm.at[pl.ds(row0 + NS * rpt, rem)],
                    acc.at[pl.ds(NS * rpt, rem)],
                )

            ips(0, 0)
            ipd(0, 0)
            ips(1, 1)
            ipd(1, 1)
            ips(2, 2)
            iws(0)
            gis(0)
            iws(1)
            gis(1)
            plsc.subcore_barrier()

            def body(v, u, w, first=False):
                gw(u)
                ips(v + 3, u)
                iwd(u)
                sis(u)
                if not first:
                    sw(w)
                ipd(v + 2, w)
                iws(w)
                gis(w)

            body(0, 0, 2, first=True)

            def triple(j, carry):
                for k, (u, w) in enumerate(((1, 0), (2, 1), (0, 2))):
                    body(1 + 3 * j + k, u, w)
                return carry

            lax.fori_loop(0, 26, triple, 0)
            body(79, 1, 0)
            gw(2)
            gw(0)
            sw(1)
            iws(1)
            iwd(2)
            iwd(0)
            plsc.subcore_barrier()
            pltpu.sync_copy(
                acc.at[pl.ds(s * rpt, rpt)],
                out_hbm.at[pl.ds(row0 + s * rpt, rpt)],
            )

            @pl.when(s == NS - 1)
            def _():
                pltpu.sync_copy(
                    acc.at[pl.ds(NS * rpt, rem)],
                    out_hbm.at[pl.ds(row0 + NS * rpt, rem)],
                )

            if r != cpc - 1:
                plsc.subcore_barrier()

    return prop_kernel


_deg_call = _make_deg()
_prop4_call = _make_prop(4)
_prop2_call = _make_prop(2)


def _dis_body(degp_ref, dis_ref):
    d = 1.0 + degp_ref[0:NDEG // F, :] + degp_ref[NDEG // F:, :]
    dis_ref[...] = lax.rsqrt(d)


def _dis_call(degp):
    out = pl.pallas_call(
        _dis_body,
        out_shape=jax.ShapeDtypeStruct((NDEG // F, F), jnp.float32),
    )(degp.reshape(2 * NDEG // F, F))
    return out.reshape(NDEG)[:N].reshape(N, 1)


def _mm_scale_body(x_ref, w_ref, dis_ref, out_ref):
    h = jnp.dot(x_ref[...], w_ref[...], preferred_element_type=jnp.float32)
    out_ref[...] = h * dis_ref[...]


def _mm_scale_call(x, W, dis2d, C):
    k = x.shape[1]
    return pl.pallas_call(
        _mm_scale_body,
        grid=(N // BN, C),
        in_specs=[
            pl.BlockSpec((BN, k), lambda n, c: (n, 0)),
            pl.BlockSpec((k, F), lambda n, c: (0, c)),
            pl.BlockSpec((BN, 1), lambda n, c: (n, 0)),
        ],
        out_specs=pl.BlockSpec((BN, F), lambda n, c: (c * (N // BN) + n, 0)),
        out_shape=jax.ShapeDtypeStruct((C * N, F), jnp.float32),
    )(x, W, dis2d)


def _mid_body(p_ref, b_ref, w_ref, dis_ref, out_ref, *, nk):
    k = pl.program_id(2)

    @pl.when(k == 0)
    def _():
        out_ref[...] = jnp.zeros_like(out_ref)

    t = jnp.maximum(p_ref[...] * dis_ref[...] + b_ref[0], 0.0)
    out_ref[...] += jnp.dot(t, w_ref[...], preferred_element_type=jnp.float32)

    @pl.when(k == nk - 1)
    def _():
        out_ref[...] *= dis_ref[...]


def _mid_call(p1, b1r, W2, dis2d, C_in, C_out):
    return pl.pallas_call(
        functools.partial(_mid_body, nk=C_in),
        grid=(N // BN, C_out, C_in),
        in_specs=[
            pl.BlockSpec((BN, F), lambda n, f, k: (k * (N // BN) + n, 0)),
            pl.BlockSpec((1, 1, F), lambda n, f, k: (k, 0, 0)),
            pl.BlockSpec((F, F), lambda n, f, k: (k, f)),
            pl.BlockSpec((BN, 1), lambda n, f, k: (n, 0)),
        ],
        out_specs=pl.BlockSpec((BN, F), lambda n, f, k: (f * (N // BN) + n, 0)),
        out_shape=jax.ShapeDtypeStruct((C_out * N, F), jnp.float32),
    )(p1, b1r, W2, dis2d)


def _final_body(p_ref, b_ref, dis_ref, out_ref):
    out_ref[...] = p_ref[...] * dis_ref[...] + b_ref[0]


def _final_call(p2, b2r, dis2d, C):
    return pl.pallas_call(
        _final_body,
        grid=(N // BN, C),
        in_specs=[
            pl.BlockSpec((BN, F), lambda n, f: (f * (N // BN) + n, 0)),
            pl.BlockSpec((1, 1, F), lambda n, f: (f, 0, 0)),
            pl.BlockSpec((BN, 1), lambda n, f: (n, 0)),
        ],
        out_specs=pl.BlockSpec((BN, F), lambda n, f: (n, f)),
        out_shape=jax.ShapeDtypeStruct((N, C * F), jnp.float32),
    )(p2, b2r, dis2d)


def kernel(x, edge_index, W1, b1, W2, b2):
    src = edge_index[0].astype(jnp.int32)
    dst = edge_index[1].astype(jnp.int32)
    src_p = jnp.concatenate(
        [src, jnp.zeros(EP - E, jnp.int32)]).reshape(NBP, EB)
    dst_p = jnp.concatenate(
        [dst, jnp.full(EP - E, N, jnp.int32)]).reshape(NBP, EB)
    src_pc4 = (src_p[None, :, :]
               + (jnp.arange(4, dtype=jnp.int32) * N)[:, None, None]
               ).reshape(4 * NBP, EB)
    pad8 = jnp.zeros((8, EB), jnp.int32)
    srcf4 = jnp.concatenate([src_pc4, pad8]).reshape(-1)
    srcf2 = jnp.concatenate([src_pc4[:2 * NBP], pad8]).reshape(-1)
    dstf = jnp.concatenate(
        [dst_p, jnp.full((8, EB), N, jnp.int32)]).reshape(-1)

    degp = _deg_call(dst)
    dis2d = _dis_call(degp)

    hs1 = _mm_scale_call(x, W1, dis2d, 4)
    p1 = _prop4_call(hs1, srcf4, dstf)
    hs2 = _mid_call(p1, b1.reshape(4, 1, F), W2, dis2d, 4, 2)
    p2 = _prop2_call(hs2, srcf2, dstf)
    z = _final_call(p2, b2.reshape(2, 1, F), dis2d, 2)
    return z
